# Initial kernel scaffold; baseline (speedup 1.0000x reference)
import jax
import jax.numpy as jnp
from jax.experimental import pallas as pl


def kernel(Y, X, edge_weight, deg, alp, lam, edge_index):
    raise NotImplementedError("write your pallas kernel here")


# speedup vs baseline: 5.0211x; 5.0211x over previous
import dataclasses
import functools

import jax
import jax.numpy as jnp
from jax import lax
from jax.experimental import pallas as pl
from jax.experimental.pallas import tpu as pltpu
from jax.experimental.pallas import tpu_sc as plsc

NC = 2
NS = 16
LN = 16
CH = 128


def _lane_splat(vec, i):
    idx = jnp.full((LN, 1), i, jnp.int32)
    dn = lax.GatherDimensionNumbers(
        offset_dims=(), collapsed_slice_dims=(0,), start_index_map=(0,))
    return lax.gather(vec, idx, dn, slice_sizes=(1,),
                      mode=lax.GatherScatterMode.PROMISE_IN_BOUNDS)


def _norm_body(deg_ref, lam_ref, nh_ref):
    lam = lam_ref[0, 0]
    dl = lam * deg_ref[...] + (1.0 - lam)
    nh_ref[...] = lax.rsqrt(dl)


def _combine_body(y_ref, x_ref, deg_ref, h_ref, alp_ref, lam_ref, o_ref):
    alp = alp_ref[0, 0]
    lam = lam_ref[0, 0]
    dl = lam * deg_ref[...] + (1.0 - lam)
    nh = lax.rsqrt(dl)
    agg = jnp.concatenate([h_ref[0], h_ref[1]], axis=1)
    o_ref[...] = ((1.0 - alp) * y_ref[...]
                  + (alp * lam) * (nh * agg)
                  + alp * (x_ref[...] / dl))


def _make_sc_kernel(n2, dh, chunks, npad):
    rows_per_tile = n2 // NS
    mesh = plsc.VectorSubcoreMesh(core_axis_name="c", subcore_axis_name="s")
    cp = pltpu.CompilerParams()
    for field, val in (("needs_layout_passes", False),
                       ("use_tc_tiling_on_sc", False)):
        if field in pltpu.CompilerParams.__dataclass_fields__:
            cp = dataclasses.replace(cp, **{field: val})

    @functools.partial(
        pl.kernel,
        mesh=mesh,
        compiler_params=cp,
        out_type=jax.ShapeDtypeStruct((NC, n2, dh), jnp.float32),
        scratch_types=[
            pltpu.VMEM((chunks, CH), jnp.int32),
            pltpu.VMEM((chunks, CH), jnp.int32),
            pltpu.VMEM((chunks, CH), jnp.float32),
            pltpu.VMEM((npad,), jnp.float32),
            pltpu.VMEM((CH, dh), jnp.float32),
            pltpu.VMEM((LN,), jnp.float32),
            pltpu.VMEM_SHARED((n2, dh), jnp.float32),
        ],
    )
    def sc_fn(yh, srcs, dsts, ws, nh, out,
              src_v, dst_v, w_v, nh_v, rows_v, sv_v, acc):
        c = lax.axis_index("c")
        s = lax.axis_index("s")
        base = s * rows_per_tile

        pltpu.sync_copy(srcs.at[s], src_v)
        pltpu.sync_copy(dsts.at[s], dst_v)
        pltpu.sync_copy(ws.at[s], w_v)
        pltpu.sync_copy(nh, nh_v)

        @pl.loop(0, CH)
        def _zero_row(r):
            for j in range(dh // LN):
                rows_v[r, pl.ds(j * LN, LN)] = jnp.zeros((LN,), jnp.float32)

        n_full, rem = divmod(rows_per_tile, CH)
        for k in range(n_full):
            pltpu.sync_copy(rows_v, acc.at[pl.ds(base + k * CH, CH)])
        if rem:
            pltpu.sync_copy(rows_v.at[pl.ds(0, rem)],
                            acc.at[pl.ds(base + n_full * CH, rem)])

        plsc.subcore_barrier()

        @pl.loop(0, chunks)
        def _chunk(ci):
            pltpu.sync_copy(yh.at[c].at[src_v.at[ci]], rows_v)
            for g in range(CH // LN):
                sidx = src_v[ci, pl.ds(g * LN, LN)]
                wv = w_v[ci, pl.ds(g * LN, LN)]
                nh16 = plsc.load_gather(nh_v, [sidx])
                sv = wv * nh16
                for i in range(LN):
                    sp = _lane_splat(sv, i)
                    e = g * LN + i
                    for j in range(dh // LN):
                        slc = pl.ds(j * LN, LN)
                        rows_v[e, slc] = rows_v[e, slc] * sp
            pltpu.sync_copy(rows_v, acc.at[dst_v.at[ci]], add=True)

        plsc.subcore_barrier()
        pltpu.sync_copy(acc.at[pl.ds(base, rows_per_tile)],
                        out.at[c, pl.ds(base, rows_per_tile)])

    return sc_fn


def kernel(Y, X, edge_weight, deg, alp, lam, edge_index):
    n, d = Y.shape
    e = edge_weight.shape[0]
    dh = d // 2
    chunks = -(-e // (NS * CH))
    epad = NS * chunks * CH
    npad = -(-n // 128) * 128
    n2 = NS * 8 * (-(-n // (NS * 8)))

    src = edge_index[0].astype(jnp.int32)
    dst = edge_index[1].astype(jnp.int32)
    w = edge_weight.astype(jnp.float32)
    pad = epad - e
    if pad:
        src = jnp.concatenate([src, jnp.zeros((pad,), jnp.int32)])
        dst = jnp.concatenate([dst, jnp.zeros((pad,), jnp.int32)])
        w = jnp.concatenate([w, jnp.zeros((pad,), jnp.float32)])
    src3 = src.reshape(NS, chunks, CH)
    dst3 = dst.reshape(NS, chunks, CH)
    w3 = w.reshape(NS, chunks, CH)
    ypad = Y
    if n2 > n:
        ypad = jnp.concatenate([Y, jnp.zeros((n2 - n, d), jnp.float32)])
    yh = jnp.stack([ypad[:, :dh], ypad[:, dh:]])

    deg_pad = deg
    if npad > n:
        deg_pad = jnp.concatenate([deg, jnp.ones((npad - n,), jnp.float32)])
    lam11 = lam.reshape(1, 1)
    alp11 = alp.reshape(1, 1)

    nh_pad = pl.pallas_call(
        _norm_body,
        out_shape=jax.ShapeDtypeStruct((npad // 128, 128), jnp.float32),
    )(deg_pad.reshape(npad // 128, 128), lam11)
    nh_flat = nh_pad.reshape(npad)

    halves = _make_sc_kernel(n2, dh, chunks, npad)(yh, src3, dst3, w3,
                                                   nh_flat)[:, :n, :]

    blk = 2000
    out = pl.pallas_call(
        _combine_body,
        grid=(n // blk,),
        in_specs=[
            pl.BlockSpec((blk, d), lambda i: (i, 0)),
            pl.BlockSpec((blk, d), lambda i: (i, 0)),
            pl.BlockSpec((blk, 1), lambda i: (i, 0)),
            pl.BlockSpec((NC, blk, dh), lambda i: (0, i, 0)),
            pl.BlockSpec((1, 1), lambda i: (0, 0)),
            pl.BlockSpec((1, 1), lambda i: (0, 0)),
        ],
        out_specs=pl.BlockSpec((blk, d), lambda i: (i, 0)),
        out_shape=jax.ShapeDtypeStruct((n, d), jnp.float32),
    )(Y, X, deg[:, None], halves, alp11, lam11)
    return out
